# Initial kernel scaffold; baseline (speedup 1.0000x reference)
import jax
import jax.numpy as jnp
from jax.experimental import pallas as pl


def kernel(x, Wg, bg, W1, b1, W2, b2):
    raise NotImplementedError("write your pallas kernel here")


# speedup vs baseline: 1.1618x; 1.1618x over previous
import functools
import numpy as np
import jax, jax.numpy as jnp
from jax.experimental import pallas as pl
from jax.experimental.pallas import tpu as pltpu

S, D, H, E, K = 2048, 1024, 2048, 8, 2
BLK = 256
NB = (S * K) // BLK + E
NP = NB * BLK


def _gating_body(x_ref, wg_ref, bg_ref, logits_ref, idx_ref, w_ref):
    x = x_ref[...]
    lg = jax.lax.dot_general(x, wg_ref[...], (((1,), (0,)), ((), ())),
                             preferred_element_type=jnp.float32,
                             precision=jax.lax.Precision.DEFAULT)
    lg = lg + bg_ref[...]
    logits_ref[...] = lg
    ii = jax.lax.broadcasted_iota(jnp.int32, lg.shape, 1)
    m0 = jnp.max(lg, axis=1, keepdims=True)
    i0 = jnp.min(jnp.where(lg == m0, ii, E), axis=1, keepdims=True)
    lg2 = jnp.where(ii == i0, -jnp.inf, lg)
    m1 = jnp.max(lg2, axis=1, keepdims=True)
    i1 = jnp.min(jnp.where(lg2 == m1, ii, E), axis=1, keepdims=True)
    t = jnp.exp(m1 - m0)
    w0 = 1.0 / (1.0 + t)
    w1 = t / (1.0 + t)
    idx_ref[...] = jnp.concatenate([i0, i1], axis=1)
    w_ref[...] = jnp.concatenate([w0, w1], axis=1)


def _gating(x_flat, Wg, bg):
    return pl.pallas_call(
        _gating_body,
        out_shape=(
            jax.ShapeDtypeStruct((S, E), jnp.float32),
            jax.ShapeDtypeStruct((S, K), jnp.int32),
            jax.ShapeDtypeStruct((S, K), jnp.float32),
        ),
    )(x_flat, Wg, bg.reshape(1, E))


def _ffn_body(be_ref, nact_ref, xs_ref, w1_ref, b1_ref, w2_ref, b2_ref, out_ref):
    @pl.when(pl.program_id(0) < nact_ref[0])
    def _():
        h = jax.lax.dot_general(xs_ref[...], w1_ref[0], (((1,), (0,)), ((), ())),
                                preferred_element_type=jnp.float32,
                                precision=jax.lax.Precision.HIGHEST)
        h = h + b1_ref[0]
        h = 0.5 * h * (1.0 + jax.lax.erf(h / np.sqrt(2).astype(np.float32)))
        o = jax.lax.dot_general(h, w2_ref[0], (((1,), (0,)), ((), ())),
                                preferred_element_type=jnp.float32,
                                precision=jax.lax.Precision.HIGHEST)
        out_ref[...] = o + b2_ref[0]


def _ffn(xs, W1, b1, W2, b2, blk_expert, nact):
    grid_spec = pltpu.PrefetchScalarGridSpec(
        num_scalar_prefetch=2,
        grid=(NB,),
        in_specs=[
            pl.BlockSpec((BLK, D), lambda i, be, na: (i, 0)),
            pl.BlockSpec((1, D, H), lambda i, be, na: (be[i], 0, 0)),
            pl.BlockSpec((1, 1, H), lambda i, be, na: (be[i], 0, 0)),
            pl.BlockSpec((1, H, D), lambda i, be, na: (be[i], 0, 0)),
            pl.BlockSpec((1, 1, D), lambda i, be, na: (be[i], 0, 0)),
        ],
        out_specs=pl.BlockSpec((BLK, D), lambda i, be, na: (i, 0)),
    )
    return pl.pallas_call(
        _ffn_body,
        grid_spec=grid_spec,
        out_shape=jax.ShapeDtypeStruct((NP, D), jnp.float32),
        compiler_params=pltpu.CompilerParams(
            dimension_semantics=("arbitrary",)),
    )(blk_expert, nact, xs, W1, b1.reshape(E, 1, H), W2, b2.reshape(E, 1, D))


def _route(idx):
    e = idx.reshape(-1)
    oh = jax.nn.one_hot(e, E, dtype=jnp.int32)
    counts = oh.sum(axis=0)
    rank = (jnp.cumsum(oh, axis=0) - oh)[jnp.arange(S * K), e]
    blocks_per_e = (counts + BLK - 1) // BLK
    blk_start_e = jnp.cumsum(blocks_per_e) - blocks_per_e
    pos = blk_start_e[e] * BLK + rank
    nact = jnp.sum(blocks_per_e)
    cumblocks = jnp.cumsum(blocks_per_e)
    bids = jnp.arange(NB, dtype=jnp.int32)
    blk_expert = jnp.minimum(
        jnp.searchsorted(cumblocks, bids, side="right").astype(jnp.int32), E - 1)
    sorted_token = jnp.zeros((NP,), jnp.int32).at[pos].set(
        jnp.arange(S * K, dtype=jnp.int32) // K)
    return pos.reshape(S, K), blk_expert, nact.reshape(1).astype(jnp.int32), sorted_token


def kernel(x, Wg, bg, W1, b1, W2, b2):
    Bx, Sx, Dx = x.shape
    x_flat = x.reshape(-1, Dx)
    logits, idx, w = _gating(x_flat, Wg, bg)
    pos, blk_expert, nact, sorted_token = _route(idx)
    xs = x_flat[sorted_token]
    ys = _ffn(xs, W1, b1, W2, b2, blk_expert, nact)
    y = w[:, 0:1] * ys[pos[:, 0]] + w[:, 1:2] * ys[pos[:, 1]]
    return (y.reshape(Bx, Sx, Dx), logits.reshape(Bx, Sx, E),
            idx.reshape(Bx, Sx, K))


# speedup vs baseline: 2.1754x; 1.8724x over previous
import functools
import numpy as np
import jax, jax.numpy as jnp
from jax.experimental import pallas as pl
from jax.experimental.pallas import tpu as pltpu

S, D, H, E, K = 2048, 1024, 2048, 8, 2
BLK = 256
NB = (S * K) // BLK + E
NP = NB * BLK


def _gating_body(x_ref, wg_ref, bg_ref, logits_ref, idx_ref, w_ref):
    x = x_ref[...]
    lg = jax.lax.dot_general(x, wg_ref[...], (((1,), (0,)), ((), ())),
                             preferred_element_type=jnp.float32,
                             precision=jax.lax.Precision.DEFAULT)
    lg = lg + bg_ref[...]
    logits_ref[...] = lg
    ii = jax.lax.broadcasted_iota(jnp.int32, lg.shape, 1)
    m0 = jnp.max(lg, axis=1, keepdims=True)
    i0 = jnp.min(jnp.where(lg == m0, ii, E), axis=1, keepdims=True)
    lg2 = jnp.where(ii == i0, -jnp.inf, lg)
    m1 = jnp.max(lg2, axis=1, keepdims=True)
    i1 = jnp.min(jnp.where(lg2 == m1, ii, E), axis=1, keepdims=True)
    t = jnp.exp(m1 - m0)
    w0 = 1.0 / (1.0 + t)
    w1 = t / (1.0 + t)
    idx_ref[...] = jnp.concatenate([i0, i1], axis=1)
    w_ref[...] = jnp.concatenate([w0, w1], axis=1)


def _gating(x_flat, Wg, bg):
    return pl.pallas_call(
        _gating_body,
        out_shape=(
            jax.ShapeDtypeStruct((S, E), jnp.float32),
            jax.ShapeDtypeStruct((S, K), jnp.int32),
            jax.ShapeDtypeStruct((S, K), jnp.float32),
        ),
    )(x_flat, Wg, bg.reshape(1, E))


def _ffn_body(be_ref, nact_ref, xs_ref, w1_ref, b1_ref, w2_ref, b2_ref, out_ref):
    @pl.when(pl.program_id(0) < nact_ref[0])
    def _():
        h = jax.lax.dot_general(xs_ref[...], w1_ref[0], (((1,), (0,)), ((), ())),
                                preferred_element_type=jnp.float32,
                                precision=jax.lax.Precision.DEFAULT)
        h = h + b1_ref[0]
        h = 0.5 * h * (1.0 + jax.lax.erf(h / np.sqrt(2).astype(np.float32)))
        o = jax.lax.dot_general(h, w2_ref[0], (((1,), (0,)), ((), ())),
                                preferred_element_type=jnp.float32,
                                precision=jax.lax.Precision.DEFAULT)
        out_ref[...] = o + b2_ref[0]


def _ffn(xs, W1, b1, W2, b2, blk_expert, nact):
    grid_spec = pltpu.PrefetchScalarGridSpec(
        num_scalar_prefetch=2,
        grid=(NB,),
        in_specs=[
            pl.BlockSpec((BLK, D), lambda i, be, na: (i, 0)),
            pl.BlockSpec((1, D, H), lambda i, be, na: (be[i], 0, 0)),
            pl.BlockSpec((1, 1, H), lambda i, be, na: (be[i], 0, 0)),
            pl.BlockSpec((1, H, D), lambda i, be, na: (be[i], 0, 0)),
            pl.BlockSpec((1, 1, D), lambda i, be, na: (be[i], 0, 0)),
        ],
        out_specs=pl.BlockSpec((BLK, D), lambda i, be, na: (i, 0)),
    )
    return pl.pallas_call(
        _ffn_body,
        grid_spec=grid_spec,
        out_shape=jax.ShapeDtypeStruct((NP, D), jnp.float32),
        compiler_params=pltpu.CompilerParams(
            dimension_semantics=("arbitrary",)),
    )(blk_expert, nact, xs, W1, b1.reshape(E, 1, H), W2, b2.reshape(E, 1, D))


def _route(idx):
    e = idx.reshape(-1)
    oh = jax.nn.one_hot(e, E, dtype=jnp.int32)
    counts = oh.sum(axis=0)
    rank = (jnp.cumsum(oh, axis=0) - oh)[jnp.arange(S * K), e]
    blocks_per_e = (counts + BLK - 1) // BLK
    blk_start_e = jnp.cumsum(blocks_per_e) - blocks_per_e
    pos = blk_start_e[e] * BLK + rank
    nact = jnp.sum(blocks_per_e)
    cumblocks = jnp.cumsum(blocks_per_e)
    bids = jnp.arange(NB, dtype=jnp.int32)
    blk_expert = jnp.minimum(
        jnp.searchsorted(cumblocks, bids, side="right").astype(jnp.int32), E - 1)
    sorted_token = jnp.zeros((NP,), jnp.int32).at[pos].set(
        jnp.arange(S * K, dtype=jnp.int32) // K)
    return pos.reshape(S, K), blk_expert, nact.reshape(1).astype(jnp.int32), sorted_token


def kernel(x, Wg, bg, W1, b1, W2, b2):
    Bx, Sx, Dx = x.shape
    x_flat = x.reshape(-1, Dx)
    logits, idx, w = _gating(x_flat, Wg, bg)
    pos, blk_expert, nact, sorted_token = _route(idx)
    xs = x_flat[sorted_token]
    ys = _ffn(xs, W1, b1, W2, b2, blk_expert, nact)
    y = w[:, 0:1] * ys[pos[:, 0]] + w[:, 1:2] * ys[pos[:, 1]]
    return (y.reshape(Bx, Sx, Dx), logits.reshape(Bx, Sx, E),
            idx.reshape(Bx, Sx, K))
